# baseline (speedup 1.0000x reference)
import functools

import jax
import jax.numpy as jnp
from jax import lax
from jax.experimental import pallas as pl
from jax.experimental.pallas import tpu as pltpu
from jax.experimental.pallas import tpu_sc as plsc

_NSC = 2
_NTILE = 16
_LANES = 16

_EPS = 1e-5


def _sc_mesh():
    return plsc.VectorSubcoreMesh(core_axis_name="c", subcore_axis_name="s",
                                  num_cores=_NSC, num_subcores=_NTILE)


def _deg_dinv(dstloc, half, rpt, ept, kw):
    npad = _NTILE * rpt
    nw = ept // kw

    def body(dst_hbm, dinv_hbm, deg_sh, onesv, idxd, degv, dinvv):
        c = lax.axis_index("c")
        s = lax.axis_index("s")
        zv = jnp.zeros((16,), jnp.float32)
        ov = jnp.ones((16,), jnp.float32)

        def z_body(i, carry):
            degv[pl.ds(i * 16, 16)] = zv
            return carry

        lax.fori_loop(0, rpt // 16, z_body, 0)

        def o_body(i, carry):
            onesv[pl.ds(i * 16, 16)] = ov
            return carry

        lax.fori_loop(0, kw // 16, o_body, 0)
        pltpu.sync_copy(degv, deg_sh.at[pl.ds(s * rpt, rpt)])
        plsc.subcore_barrier()
        ebase = c * (ept * _NTILE) + s * ept

        def w_body(w, carry):
            base = ebase + w * kw
            pltpu.sync_copy(dst_hbm.at[pl.ds(base, kw)], idxd)
            pltpu.sync_copy(onesv, deg_sh.at[idxd], add=True)
            return carry

        lax.fori_loop(0, nw, w_body, 0)
        plsc.subcore_barrier()
        pltpu.sync_copy(deg_sh.at[pl.ds(s * rpt, rpt)], degv)

        def r_body(i, carry):
            x = degv[pl.ds(i * 16, 16)] + 1.0
            xi = plsc.bitcast(x, jnp.int32)
            y = plsc.bitcast(jnp.int32(0x5F3759DF) - (xi >> 1), jnp.float32)
            y = y * (1.5 - 0.5 * x * y * y)
            y = y * (1.5 - 0.5 * x * y * y)
            y = y * (1.5 - 0.5 * x * y * y)
            dinvv[pl.ds(i * 16, 16)] = y
            return carry

        lax.fori_loop(0, rpt // 16, r_body, 0)
        pltpu.sync_copy(dinvv, dinv_hbm.at[pl.ds(c * npad + s * rpt, rpt)])

    fn = pl.kernel(
        body,
        out_type=jax.ShapeDtypeStruct((_NSC * npad,), jnp.float32),
        mesh=_sc_mesh(),
        compiler_params=pltpu.CompilerParams(needs_layout_passes=False),
        scratch_types=[
            pltpu.VMEM_SHARED((npad,), jnp.float32),
            pltpu.VMEM((kw,), jnp.float32),
            pltpu.VMEM((kw,), jnp.int32),
            pltpu.VMEM((rpt,), jnp.float32),
            pltpu.VMEM((rpt,), jnp.float32),
        ],
    )
    return fn(dstloc)


def _scatter_rows(h, src, dstloc, d, half, rpt, ept, kw):
    nw = ept // kw

    def body(h_hbm, src_hbm, dst_hbm, agg_hbm,
             h_sh, agg_sh, rows0, rows1, s0, s1, d0, d1,
             sem0, sem1, sems0, sems1, semd0, semd1):
        c = lax.axis_index("c")
        s = lax.axis_index("s")
        zv = jnp.zeros((16,), jnp.float32)
        ebase = (c * _NTILE + s) * ept
        hoff = jnp.minimum(s * rpt, half - rpt)

        def iload(w, hbm, buf, sem):
            pltpu.async_copy(hbm.at[pl.ds(ebase + w * kw, kw)], buf, sem)

        def iwait(w, hbm, buf, sem):
            pltpu.make_async_copy(hbm.at[pl.ds(ebase + w * kw, kw)],
                                  buf, sem).wait()

        iload(0, src_hbm, s0, sems0)
        iload(1, src_hbm, s1, sems1)
        iload(0, dst_hbm, d0, semd0)
        iload(1, dst_hbm, d1, semd1)

        def z_body(i, carry):
            for j in range(d // 16):
                rows0[i, pl.ds(j * 16, 16)] = zv
            return carry

        lax.fori_loop(0, kw, z_body, 0)
        for i in range(-(-rpt // kw)):
            o = min(i * kw, rpt - kw)
            pltpu.sync_copy(rows0, agg_sh.at[pl.ds(hoff + o, kw)])
        for i in range(-(-rpt // kw)):
            o = min(i * kw, rpt - kw)
            pltpu.sync_copy(h_hbm.at[pl.ds(c * half + hoff + o, kw)], rows0)
            pltpu.sync_copy(rows0, h_sh.at[pl.ds(hoff + o, kw)])
        plsc.subcore_barrier()

        def g(sbuf, buf, sem):
            pltpu.async_copy(h_sh.at[sbuf], buf, sem)

        def gwait(sbuf, buf, sem):
            pltpu.make_async_copy(h_sh.at[sbuf], buf, sem).wait()

        def sadd(dbuf, buf):
            pltpu.sync_copy(buf, agg_sh.at[dbuf], add=True)

        iwait(0, src_hbm, s0, sems0)
        g(s0, rows0, sem0)

        def pair(wp, carry):
            w = 2 * wp
            iwait(w + 1, src_hbm, s1, sems1)
            g(s1, rows1, sem1)
            gwait(s0, rows0, sem0)
            iwait(w, dst_hbm, d0, semd0)

            @pl.when(w + 2 < nw)
            def _():
                iload(w + 2, src_hbm, s0, sems0)
                iload(w + 2, dst_hbm, d0, semd0)

            sadd(d0, rows0)

            @pl.when(w + 2 < nw)
            def _():
                iwait(w + 2, src_hbm, s0, sems0)
                g(s0, rows0, sem0)

            gwait(s1, rows1, sem1)
            iwait(w + 1, dst_hbm, d1, semd1)

            @pl.when(w + 3 < nw)
            def _():
                iload(w + 3, src_hbm, s1, sems1)
                iload(w + 3, dst_hbm, d1, semd1)

            sadd(d1, rows1)
            return carry

        lax.fori_loop(0, nw // 2, pair, 0)
        if nw % 2 == 1:
            gwait(s0, rows0, sem0)
            iwait(nw - 1, dst_hbm, d0, semd0)
            sadd(d0, rows0)
        plsc.subcore_barrier()
        pltpu.sync_copy(agg_sh.at[pl.ds(hoff, rpt)],
                        agg_hbm.at[c, pl.ds(hoff, rpt)])

    fn = pl.kernel(
        body,
        out_type=jax.ShapeDtypeStruct((_NSC, half, d), jnp.float32),
        mesh=_sc_mesh(),
        compiler_params=pltpu.CompilerParams(needs_layout_passes=False),
        scratch_types=[
            pltpu.VMEM_SHARED((half, d), jnp.float32),
            pltpu.VMEM_SHARED((half, d), jnp.float32),
            pltpu.VMEM((kw, d), jnp.float32),
            pltpu.VMEM((kw, d), jnp.float32),
            pltpu.VMEM((kw,), jnp.int32),
            pltpu.VMEM((kw,), jnp.int32),
            pltpu.VMEM((kw,), jnp.int32),
            pltpu.VMEM((kw,), jnp.int32),
            pltpu.SemaphoreType.DMA,
            pltpu.SemaphoreType.DMA,
            pltpu.SemaphoreType.DMA,
            pltpu.SemaphoreType.DMA,
            pltpu.SemaphoreType.DMA,
            pltpu.SemaphoreType.DMA,
        ],
    )
    return fn(h, src, dstloc)


def _mm_body(x_ref, dinv_ref, w_ref, o_ref):
    o_ref[...] = jnp.dot(x_ref[...] * dinv_ref[...], w_ref[...],
                         preferred_element_type=jnp.float32)


def _mm1(x, dinv, w, blk):
    bn, d = x.shape
    grid = (bn // blk,)
    return pl.pallas_call(
        _mm_body,
        grid=grid,
        in_specs=[
            pl.BlockSpec((blk, d), lambda i: (i, 0)),
            pl.BlockSpec((blk, 1), lambda i: (i, 0)),
            pl.BlockSpec((d, d), lambda i: (0, 0)),
        ],
        out_specs=pl.BlockSpec((blk, d), lambda i: (i, 0)),
        out_shape=jax.ShapeDtypeStruct((bn, d), jnp.float32),
    )(x, dinv, w)


def _gcn_post(x, h, agg, dinv, b, g, be):
    z = dinv * (agg + h) + b
    mu = jnp.mean(z, axis=1, keepdims=True)
    var = jnp.mean((z - mu) ** 2, axis=1, keepdims=True)
    hn = (z - mu) * lax.rsqrt(var + _EPS) * g + be
    o = hn + x
    return o * jax.nn.sigmoid(o)


def _epi_body(x_ref, h_ref, agg_ref, dinv_ref, b_ref, g_ref, be_ref, w2_ref,
              x2_ref, h2_ref):
    dv = dinv_ref[...]
    y = _gcn_post(x_ref[...], h_ref[...], agg_ref[...], dv,
                  b_ref[...], g_ref[...], be_ref[...])
    x2_ref[...] = y
    h2_ref[...] = jnp.dot(y * dv, w2_ref[...],
                          preferred_element_type=jnp.float32)


def _epi_mm2(x, h, agg, dinv, b, g, be, w2, blk):
    bn, d = x.shape
    grid = (bn // blk,)
    row = lambda i: (i, 0)
    fixed = lambda i: (0, 0)
    return pl.pallas_call(
        _epi_body,
        grid=grid,
        in_specs=[
            pl.BlockSpec((blk, d), row),
            pl.BlockSpec((blk, d), row),
            pl.BlockSpec((blk, d), row),
            pl.BlockSpec((blk, 1), row),
            pl.BlockSpec((1, d), fixed),
            pl.BlockSpec((1, d), fixed),
            pl.BlockSpec((1, d), fixed),
            pl.BlockSpec((d, d), fixed),
        ],
        out_specs=[pl.BlockSpec((blk, d), row), pl.BlockSpec((blk, d), row)],
        out_shape=[jax.ShapeDtypeStruct((bn, d), jnp.float32),
                   jax.ShapeDtypeStruct((bn, d), jnp.float32)],
    )(x, h, agg, dinv, b, g, be, w2)


def _fin_body(x_ref, h_ref, agg_ref, dinv_ref, b_ref, g_ref, be_ref, o_ref):
    y = _gcn_post(x_ref[0], h_ref[0], agg_ref[0], dinv_ref[0],
                  b_ref[...], g_ref[...], be_ref[...])
    o_ref[...] = jnp.mean(y, axis=0, keepdims=True)[None]


def _epi_pool(x, h, agg, dinv, b, g, be, n_per_graph):
    bn, d = x.shape
    ng = bn // n_per_graph
    n = n_per_graph
    row = lambda i: (i, 0, 0)
    fixed = lambda i: (0, 0)
    x3 = x.reshape(ng, n, d)
    h3 = h.reshape(ng, n, d)
    agg3 = agg.reshape(ng, n, d)
    dinv3 = dinv.reshape(ng, n, 1)
    out = pl.pallas_call(
        _fin_body,
        grid=(ng,),
        in_specs=[
            pl.BlockSpec((1, n, d), row),
            pl.BlockSpec((1, n, d), row),
            pl.BlockSpec((1, n, d), row),
            pl.BlockSpec((1, n, 1), row),
            pl.BlockSpec((1, d), fixed),
            pl.BlockSpec((1, d), fixed),
            pl.BlockSpec((1, d), fixed),
        ],
        out_specs=pl.BlockSpec((1, 1, d), row),
        out_shape=jax.ShapeDtypeStruct((ng, 1, d), jnp.float32),
    )(x3, h3, agg3, dinv3, b, g, be)
    return out.reshape(ng, d)


def kernel(node_feats, edge_indices, W1, b1, g1, be1, W2, b2, g2, be2):
    B, N, d = node_feats.shape
    E = edge_indices.shape[2]
    BN = B * N
    half = BN // _NSC
    rpt = (-(-half // _NTILE) + 7) // 8 * 8
    ept = (B * E) // (_NSC * _NTILE)
    kw = 80
    assert ept % kw == 0 and half % 8 == 0

    off = (jnp.arange(B, dtype=edge_indices.dtype) * N)[:, None, None]
    ei = (edge_indices + off).transpose(1, 0, 2).reshape(2, -1)
    src = ei[0].astype(jnp.int32)
    dst = ei[1].astype(jnp.int32)
    srcloc = jnp.where(src >= half, src - half, src)
    dstloc = jnp.where(dst >= half, dst - half, dst)

    x0 = node_feats.reshape(BN, d)

    npad = _NTILE * rpt
    dinv_p = _deg_dinv(dstloc, half, rpt, ept, kw)
    dinv = dinv_p.reshape(_NSC, npad)[:, :half].reshape(BN, 1)

    blk = 1000
    b1r, g1r, be1r = b1.reshape(1, d), g1.reshape(1, d), be1.reshape(1, d)
    b2r, g2r, be2r = b2.reshape(1, d), g2.reshape(1, d), be2.reshape(1, d)

    h1 = _mm1(x0, dinv, W1, blk)
    agg1 = _scatter_rows(h1, srcloc, dstloc, d, half, rpt, ept, kw)[:, :half].reshape(BN, d)
    x2, h2 = _epi_mm2(x0, h1, agg1, dinv, b1r, g1r, be1r, W2, blk)
    agg2 = _scatter_rows(h2, srcloc, dstloc, d, half, rpt, ept, kw)[:, :half].reshape(BN, d)
    return _epi_pool(x2, h2, agg2, dinv, b2r, g2r, be2r, N)


# speedup vs baseline: 1.4029x; 1.4029x over previous
import functools

import jax
import jax.numpy as jnp
from jax import lax
from jax.experimental import pallas as pl
from jax.experimental.pallas import tpu as pltpu
from jax.experimental.pallas import tpu_sc as plsc

_NSC = 2
_NTILE = 16
_LANES = 16

_EPS = 1e-5


def _sc_mesh():
    return plsc.VectorSubcoreMesh(core_axis_name="c", subcore_axis_name="s",
                                  num_cores=_NSC, num_subcores=_NTILE)


def _deg_dinv(dstloc, half, rpt, ept, kw):
    npad = _NTILE * rpt
    nw = ept // kw

    def body(dst_hbm, dinv_hbm, deg_sh, onesv, idxd, degv, dinvv):
        c = lax.axis_index("c")
        s = lax.axis_index("s")
        zv = jnp.zeros((16,), jnp.float32)
        ov = jnp.ones((16,), jnp.float32)

        def z_body(i, carry):
            degv[pl.ds(i * 16, 16)] = zv
            return carry

        lax.fori_loop(0, rpt // 16, z_body, 0)

        def o_body(i, carry):
            onesv[pl.ds(i * 16, 16)] = ov
            return carry

        lax.fori_loop(0, kw // 16, o_body, 0)
        pltpu.sync_copy(degv, deg_sh.at[pl.ds(s * rpt, rpt)])
        plsc.subcore_barrier()
        ebase = c * (ept * _NTILE) + s * ept

        def w_body(w, carry):
            base = ebase + w * kw
            pltpu.sync_copy(dst_hbm.at[pl.ds(base, kw)], idxd)
            pltpu.sync_copy(onesv, deg_sh.at[idxd], add=True)
            return carry

        lax.fori_loop(0, nw, w_body, 0)
        plsc.subcore_barrier()
        pltpu.sync_copy(deg_sh.at[pl.ds(s * rpt, rpt)], degv)

        def r_body(i, carry):
            x = degv[pl.ds(i * 16, 16)] + 1.0
            xi = plsc.bitcast(x, jnp.int32)
            y = plsc.bitcast(jnp.int32(0x5F3759DF) - (xi >> 1), jnp.float32)
            y = y * (1.5 - 0.5 * x * y * y)
            y = y * (1.5 - 0.5 * x * y * y)
            y = y * (1.5 - 0.5 * x * y * y)
            dinvv[pl.ds(i * 16, 16)] = y
            return carry

        lax.fori_loop(0, rpt // 16, r_body, 0)
        pltpu.sync_copy(dinvv, dinv_hbm.at[pl.ds(c * npad + s * rpt, rpt)])

    fn = pl.kernel(
        body,
        out_type=jax.ShapeDtypeStruct((_NSC * npad,), jnp.float32),
        mesh=_sc_mesh(),
        compiler_params=pltpu.CompilerParams(needs_layout_passes=False),
        scratch_types=[
            pltpu.VMEM_SHARED((npad,), jnp.float32),
            pltpu.VMEM((kw,), jnp.float32),
            pltpu.VMEM((kw,), jnp.int32),
            pltpu.VMEM((rpt,), jnp.float32),
            pltpu.VMEM((rpt,), jnp.float32),
        ],
    )
    return fn(dstloc)


def _scatter_rows(h, src, dstloc, d, half, rpt, ept, kw):
    nw = ept // kw

    def body(h_hbm, src_hbm, dst_hbm, agg_hbm,
             agg_sh, rows0, rows1, s0, s1, d0, d1,
             sem0, sem1, sems0, sems1, semd0, semd1):
        c = lax.axis_index("c")
        s = lax.axis_index("s")
        zv = jnp.zeros((16,), jnp.float32)
        ebase = (c * _NTILE + s) * ept
        hoff = jnp.minimum(s * rpt, half - rpt)

        def iload(w, hbm, buf, sem):
            pltpu.async_copy(hbm.at[pl.ds(ebase + w * kw, kw)], buf, sem)

        def iwait(w, hbm, buf, sem):
            pltpu.make_async_copy(hbm.at[pl.ds(ebase + w * kw, kw)],
                                  buf, sem).wait()

        iload(0, src_hbm, s0, sems0)
        iload(1, src_hbm, s1, sems1)
        iload(0, dst_hbm, d0, semd0)
        iload(1, dst_hbm, d1, semd1)

        def z_body(i, carry):
            for j in range(d // 16):
                rows0[i, pl.ds(j * 16, 16)] = zv
            return carry

        lax.fori_loop(0, kw, z_body, 0)
        for i in range(-(-rpt // kw)):
            o = min(i * kw, rpt - kw)
            pltpu.sync_copy(rows0, agg_sh.at[pl.ds(hoff + o, kw)])
        plsc.subcore_barrier()

        def g(sbuf, buf, sem):
            pltpu.async_copy(h_hbm.at[sbuf], buf, sem)

        def gwait(sbuf, buf, sem):
            pltpu.make_async_copy(h_hbm.at[sbuf], buf, sem).wait()

        def sadd(dbuf, buf):
            pltpu.sync_copy(buf, agg_sh.at[dbuf], add=True)

        iwait(0, src_hbm, s0, sems0)
        g(s0, rows0, sem0)

        def pair(wp, carry):
            w = 2 * wp
            iwait(w + 1, src_hbm, s1, sems1)
            g(s1, rows1, sem1)
            gwait(s0, rows0, sem0)
            iwait(w, dst_hbm, d0, semd0)

            @pl.when(w + 2 < nw)
            def _():
                iload(w + 2, src_hbm, s0, sems0)
                iload(w + 2, dst_hbm, d0, semd0)

            sadd(d0, rows0)

            @pl.when(w + 2 < nw)
            def _():
                iwait(w + 2, src_hbm, s0, sems0)
                g(s0, rows0, sem0)

            gwait(s1, rows1, sem1)
            iwait(w + 1, dst_hbm, d1, semd1)

            @pl.when(w + 3 < nw)
            def _():
                iload(w + 3, src_hbm, s1, sems1)
                iload(w + 3, dst_hbm, d1, semd1)

            sadd(d1, rows1)
            return carry

        lax.fori_loop(0, nw // 2, pair, 0)
        if nw % 2 == 1:
            gwait(s0, rows0, sem0)
            iwait(nw - 1, dst_hbm, d0, semd0)
            sadd(d0, rows0)
        plsc.subcore_barrier()
        pltpu.sync_copy(agg_sh.at[pl.ds(hoff, rpt)],
                        agg_hbm.at[c, pl.ds(hoff, rpt)])

    fn = pl.kernel(
        body,
        out_type=jax.ShapeDtypeStruct((_NSC, half, d), jnp.float32),
        mesh=_sc_mesh(),
        compiler_params=pltpu.CompilerParams(needs_layout_passes=False),
        scratch_types=[
            pltpu.VMEM_SHARED((half, d), jnp.float32),
            pltpu.VMEM((kw, d), jnp.float32),
            pltpu.VMEM((kw, d), jnp.float32),
            pltpu.VMEM((kw,), jnp.int32),
            pltpu.VMEM((kw,), jnp.int32),
            pltpu.VMEM((kw,), jnp.int32),
            pltpu.VMEM((kw,), jnp.int32),
            pltpu.SemaphoreType.DMA,
            pltpu.SemaphoreType.DMA,
            pltpu.SemaphoreType.DMA,
            pltpu.SemaphoreType.DMA,
            pltpu.SemaphoreType.DMA,
            pltpu.SemaphoreType.DMA,
        ],
    )
    return fn(h, src, dstloc)


def _mm_body(x_ref, dinv_ref, w_ref, o_ref):
    o_ref[...] = jnp.dot(x_ref[...] * dinv_ref[...], w_ref[...],
                         preferred_element_type=jnp.float32)


def _mm1(x, dinv, w, blk):
    bn, d = x.shape
    grid = (bn // blk,)
    return pl.pallas_call(
        _mm_body,
        grid=grid,
        in_specs=[
            pl.BlockSpec((blk, d), lambda i: (i, 0)),
            pl.BlockSpec((blk, 1), lambda i: (i, 0)),
            pl.BlockSpec((d, d), lambda i: (0, 0)),
        ],
        out_specs=pl.BlockSpec((blk, d), lambda i: (i, 0)),
        out_shape=jax.ShapeDtypeStruct((bn, d), jnp.float32),
    )(x, dinv, w)


def _gcn_post(x, h, agg, dinv, b, g, be):
    z = dinv * (agg + h) + b
    mu = jnp.mean(z, axis=1, keepdims=True)
    var = jnp.mean((z - mu) ** 2, axis=1, keepdims=True)
    hn = (z - mu) * lax.rsqrt(var + _EPS) * g + be
    o = hn + x
    return o * jax.nn.sigmoid(o)


def _epi_body(x_ref, h_ref, agg_ref, dinv_ref, b_ref, g_ref, be_ref, w2_ref,
              x2_ref, h2_ref):
    dv = dinv_ref[...]
    y = _gcn_post(x_ref[...], h_ref[...], agg_ref[...], dv,
                  b_ref[...], g_ref[...], be_ref[...])
    x2_ref[...] = y
    h2_ref[...] = jnp.dot(y * dv, w2_ref[...],
                          preferred_element_type=jnp.float32)


def _epi_mm2(x, h, agg, dinv, b, g, be, w2, blk):
    bn, d = x.shape
    grid = (bn // blk,)
    row = lambda i: (i, 0)
    fixed = lambda i: (0, 0)
    return pl.pallas_call(
        _epi_body,
        grid=grid,
        in_specs=[
            pl.BlockSpec((blk, d), row),
            pl.BlockSpec((blk, d), row),
            pl.BlockSpec((blk, d), row),
            pl.BlockSpec((blk, 1), row),
            pl.BlockSpec((1, d), fixed),
            pl.BlockSpec((1, d), fixed),
            pl.BlockSpec((1, d), fixed),
            pl.BlockSpec((d, d), fixed),
        ],
        out_specs=[pl.BlockSpec((blk, d), row), pl.BlockSpec((blk, d), row)],
        out_shape=[jax.ShapeDtypeStruct((bn, d), jnp.float32),
                   jax.ShapeDtypeStruct((bn, d), jnp.float32)],
    )(x, h, agg, dinv, b, g, be, w2)


def _fin_body(x_ref, h_ref, agg_ref, dinv_ref, b_ref, g_ref, be_ref, o_ref):
    y = _gcn_post(x_ref[0], h_ref[0], agg_ref[0], dinv_ref[0],
                  b_ref[...], g_ref[...], be_ref[...])
    o_ref[...] = jnp.mean(y, axis=0, keepdims=True)[None]


def _epi_pool(x, h, agg, dinv, b, g, be, n_per_graph):
    bn, d = x.shape
    ng = bn // n_per_graph
    n = n_per_graph
    row = lambda i: (i, 0, 0)
    fixed = lambda i: (0, 0)
    x3 = x.reshape(ng, n, d)
    h3 = h.reshape(ng, n, d)
    agg3 = agg.reshape(ng, n, d)
    dinv3 = dinv.reshape(ng, n, 1)
    out = pl.pallas_call(
        _fin_body,
        grid=(ng,),
        in_specs=[
            pl.BlockSpec((1, n, d), row),
            pl.BlockSpec((1, n, d), row),
            pl.BlockSpec((1, n, d), row),
            pl.BlockSpec((1, n, 1), row),
            pl.BlockSpec((1, d), fixed),
            pl.BlockSpec((1, d), fixed),
            pl.BlockSpec((1, d), fixed),
        ],
        out_specs=pl.BlockSpec((1, 1, d), row),
        out_shape=jax.ShapeDtypeStruct((ng, 1, d), jnp.float32),
    )(x3, h3, agg3, dinv3, b, g, be)
    return out.reshape(ng, d)


def kernel(node_feats, edge_indices, W1, b1, g1, be1, W2, b2, g2, be2):
    B, N, d = node_feats.shape
    E = edge_indices.shape[2]
    BN = B * N
    half = BN // _NSC
    rpt = (-(-half // _NTILE) + 7) // 8 * 8
    ept = (B * E) // (_NSC * _NTILE)
    kw = 200
    assert ept % kw == 0 and half % 8 == 0

    off = (jnp.arange(B, dtype=edge_indices.dtype) * N)[:, None, None]
    ei = (edge_indices + off).transpose(1, 0, 2).reshape(2, -1)
    src = ei[0].astype(jnp.int32)
    dst = ei[1].astype(jnp.int32)
    dstloc = jnp.where(dst >= half, dst - half, dst)

    x0 = node_feats.reshape(BN, d)

    npad = _NTILE * rpt
    dinv_p = _deg_dinv(dstloc, half, rpt, ept, kw)
    dinv = dinv_p.reshape(_NSC, npad)[:, :half].reshape(BN, 1)

    blk = 1000
    b1r, g1r, be1r = b1.reshape(1, d), g1.reshape(1, d), be1.reshape(1, d)
    b2r, g2r, be2r = b2.reshape(1, d), g2.reshape(1, d), be2.reshape(1, d)

    h1 = _mm1(x0, dinv, W1, blk)
    agg1 = _scatter_rows(h1, src, dstloc, d, half, rpt, ept, kw)[:, :half].reshape(BN, d)
    x2, h2 = _epi_mm2(x0, h1, agg1, dinv, b1r, g1r, be1r, W2, blk)
    agg2 = _scatter_rows(h2, src, dstloc, d, half, rpt, ept, kw)[:, :half].reshape(BN, d)
    return _epi_pool(x2, h2, agg2, dinv, b2r, g2r, be2r, N)
